# baseline (speedup 1.0000x reference)
import jax
import jax.numpy as jnp
from jax.experimental import pallas as pl
from jax.experimental.pallas import tpu as pltpu

KTAPS = 20
KB = 10
NG = 2
ROWPAD = 24
AA = 21
UU = 100
LL = 334
PP = LL - KTAPS + 1
FF = 6


def _prep_kernel(pl_ref, q_ref, r_ref, rb_ref):
    pv = pl_ref[...]
    q = q_ref[...]
    meanq = jnp.mean(q)
    eps = jnp.exp(-jnp.log(1.0 / meanq))
    m = jnp.max(pv, axis=1, keepdims=True)
    e = jnp.exp(pv - m)
    p = e / jnp.sum(e, axis=1, keepdims=True)
    ratio = jnp.maximum(p / q, eps)
    rv = jnp.log(ratio)
    r_ref[...] = rv
    rb_ref[...] = jnp.zeros((256, 256), jnp.bfloat16)
    for k1 in range(NG):
        for k2 in range(KB):
            rb_ref[k2 * ROWPAD:k2 * ROWPAD + AA,
                   k1 * 128:k1 * 128 + UU] = rv[k1 * KB + k2].astype(jnp.bfloat16)


def _conv_kernel(x_ref, rb_ref, z_ref, s_ref, col0, col1):
    @pl.when(pl.program_id(0) == 0)
    def _init():
        col0[...] = jnp.zeros_like(col0)
        col1[...] = jnp.zeros_like(col1)

    rb = rb_ref[...]
    smax = None
    for r in range(FF):
        col = (col0, col1)[r % 2]
        xt = x_ref[0, r].astype(jnp.bfloat16).T
        for k2 in range(KB):
            col[k2 * ROWPAD:k2 * ROWPAD + AA, :] = xt[:, k2:k2 + PP + KB]
        y = jax.lax.dot_general(
            col[...], rb,
            (((0,), (0,)), ((), ())),
            preferred_element_type=jnp.float32)
        z = y[0:PP, 0:UU] + y[KB:KB + PP, 128:128 + UU]
        z_ref[0, r] = z
        m = jnp.max(z, axis=0)
        smax = m if smax is None else jnp.maximum(smax, m)
    s_ref[0, 0] = smax


def kernel(X, P_logit, Q):
    T, N, F, L, A = X.shape
    B2 = T * N

    R, Rbig = pl.pallas_call(
        _prep_kernel,
        out_shape=[jax.ShapeDtypeStruct((KTAPS, A, UU), jnp.float32),
                   jax.ShapeDtypeStruct((256, 256), jnp.bfloat16)],
    )(P_logit, Q.reshape(1, A, 1))

    Z4, S3 = pl.pallas_call(
        _conv_kernel,
        grid=(B2,),
        in_specs=[
            pl.BlockSpec((1, F, L, A), lambda i: (i, 0, 0, 0)),
            pl.BlockSpec((256, 256), lambda i: (0, 0)),
        ],
        out_specs=[
            pl.BlockSpec((1, F, PP, UU), lambda i: (i, 0, 0, 0)),
            pl.BlockSpec((1, 1, UU), lambda i: (i, 0, 0)),
        ],
        out_shape=[
            jax.ShapeDtypeStruct((B2, F, PP, UU), jnp.float32),
            jax.ShapeDtypeStruct((B2, 1, UU), jnp.float32),
        ],
        scratch_shapes=[pltpu.VMEM((256, PP + KB), jnp.bfloat16),
                        pltpu.VMEM((256, PP + KB), jnp.bfloat16)],
        compiler_params=pltpu.CompilerParams(
            dimension_semantics=("parallel",)),
    )(X.reshape(B2, F, L, A), Rbig)

    S = S3.reshape(T, N, UU)
    Z = Z4.reshape(T, N, F, PP, UU)
    return (S, R, Z)


# speedup vs baseline: 1.2689x; 1.2689x over previous
import jax
import jax.numpy as jnp
from jax.experimental import pallas as pl
from jax.experimental.pallas import tpu as pltpu

KTAPS = 20
KB = 10
NG = 2
ROWPAD = 24
AA = 21
UU = 100
LL = 334
PP = LL - KTAPS + 1
FF = 6
NB = 8


def _prep_kernel(pl_ref, q_ref, r_ref, rbt_ref):
    pv = pl_ref[...]
    q = q_ref[...]
    meanq = jnp.mean(q)
    eps = jnp.exp(-jnp.log(1.0 / meanq))
    m = jnp.max(pv, axis=1, keepdims=True)
    e = jnp.exp(pv - m)
    p = e / jnp.sum(e, axis=1, keepdims=True)
    ratio = jnp.maximum(p / q, eps)
    rv = jnp.log(ratio)
    r_ref[...] = rv
    rbt_ref[...] = jnp.zeros((256, 256), jnp.bfloat16)
    for k1 in range(NG):
        for k2 in range(KB):
            rbt_ref[k1 * 128:k1 * 128 + UU,
                    k2 * ROWPAD:k2 * ROWPAD + AA] = (
                rv[k1 * KB + k2].astype(jnp.bfloat16).T)


def _conv_kernel(x_ref, rbt_ref, z_ref, s_ref, col0, col1):
    col0[...] = jnp.zeros_like(col0)
    col1[...] = jnp.zeros_like(col1)
    rbt = rbt_ref[...]
    for j in range(NB):
        smax = None
        for r in range(FF):
            col = (col0, col1)[(j * FF + r) % 2]
            xt = x_ref[0, r, :, j, :].astype(jnp.bfloat16)
            for k2 in range(KB):
                col[k2 * ROWPAD:k2 * ROWPAD + AA, :] = xt[:, k2:k2 + PP + KB]
            yt = jnp.dot(rbt, col[...],
                         preferred_element_type=jnp.float32)
            zt = yt[0:UU, 0:PP] + yt[128:128 + UU, KB:KB + PP]
            z_ref[0, r, :, j, :] = zt
            m = jnp.max(zt, axis=1)
            smax = m if smax is None else jnp.maximum(smax, m)
        s_ref[0, 0, j, :] = smax


def kernel(X, P_logit, Q):
    T, N, F, L, A = X.shape

    R, RbT = pl.pallas_call(
        _prep_kernel,
        out_shape=[jax.ShapeDtypeStruct((KTAPS, A, UU), jnp.float32),
                   jax.ShapeDtypeStruct((256, 256), jnp.bfloat16)],
    )(P_logit, Q.reshape(1, A, 1))

    Xp = jnp.transpose(X, (0, 2, 4, 1, 3))

    Zp, S4 = pl.pallas_call(
        _conv_kernel,
        grid=(T, N // NB),
        in_specs=[
            pl.BlockSpec((1, F, A, NB, L), lambda t, nb: (t, 0, 0, nb, 0)),
            pl.BlockSpec((256, 256), lambda t, nb: (0, 0)),
        ],
        out_specs=[
            pl.BlockSpec((1, F, UU, NB, PP), lambda t, nb: (t, 0, 0, nb, 0)),
            pl.BlockSpec((1, 1, NB, UU), lambda t, nb: (t, nb, 0, 0)),
        ],
        out_shape=[
            jax.ShapeDtypeStruct((T, F, UU, N, PP), jnp.float32),
            jax.ShapeDtypeStruct((T, N // NB, NB, UU), jnp.float32),
        ],
        scratch_shapes=[pltpu.VMEM((256, PP + KB), jnp.bfloat16),
                        pltpu.VMEM((256, PP + KB), jnp.bfloat16)],
        compiler_params=pltpu.CompilerParams(
            dimension_semantics=("parallel", "parallel")),
    )(Xp, RbT)

    S = S4.reshape(T, N, UU)
    Z = jnp.transpose(Zp, (0, 3, 1, 4, 2))
    return (S, R, Z)


# speedup vs baseline: 1.7799x; 1.4027x over previous
import jax
import jax.numpy as jnp
from jax.experimental import pallas as pl
from jax.experimental.pallas import tpu as pltpu

KTAPS = 20
KB = 10
NG = 2
ROWPAD = 24
AA = 21
UU = 100
LL = 334
PP = LL - KTAPS + 1
FF = 6
NB = 8


def _prep_kernel(pl_ref, q_ref, r_ref, rbt_ref):
    pv = pl_ref[...]
    q = q_ref[...]
    meanq = jnp.mean(q)
    eps = jnp.exp(-jnp.log(1.0 / meanq))
    m = jnp.max(pv, axis=1, keepdims=True)
    e = jnp.exp(pv - m)
    p = e / jnp.sum(e, axis=1, keepdims=True)
    ratio = jnp.maximum(p / q, eps)
    rv = jnp.log(ratio)
    r_ref[...] = rv
    rbt_ref[...] = jnp.zeros((256, 256), jnp.bfloat16)
    for k1 in range(NG):
        for k2 in range(KB):
            rbt_ref[k1 * 128:k1 * 128 + UU,
                    k2 * ROWPAD:k2 * ROWPAD + AA] = (
                rv[k1 * KB + k2].astype(jnp.bfloat16).T)


def _conv_kernel(x_ref, rbt_ref, z_ref, s_ref, *cols):
    for c in cols:
        c[...] = jnp.zeros_like(c)
    rbt = rbt_ref[...]

    def build(i):
        j, r = divmod(i, FF)
        xt = x_ref[0, j, r]
        col = cols[i % len(cols)]
        for k2 in range(KB):
            col[k2 * ROWPAD:k2 * ROWPAD + AA, :] = xt[:, k2:k2 + PP + KB]

    build(0)
    smax = None
    for i in range(NB * FF):
        j, r = divmod(i, FF)
        yt = jnp.dot(rbt, cols[i % len(cols)][...],
                     preferred_element_type=jnp.float32)
        if i + 1 < NB * FF:
            build(i + 1)
        zt = yt[0:UU, 0:PP] + yt[128:128 + UU, KB:KB + PP]
        z_ref[0, r, :, j, :] = zt
        m = jnp.max(zt, axis=1)
        smax = m if r == 0 else jnp.maximum(smax, m)
        if r == FF - 1:
            s_ref[0, 0, j, :] = smax


def kernel(X, P_logit, Q):
    T, N, F, L, A = X.shape

    R, RbT = pl.pallas_call(
        _prep_kernel,
        out_shape=[jax.ShapeDtypeStruct((KTAPS, A, UU), jnp.float32),
                   jax.ShapeDtypeStruct((256, 256), jnp.bfloat16)],
    )(P_logit, Q.reshape(1, A, 1))

    Xt = jnp.transpose(X, (0, 1, 2, 4, 3)).astype(jnp.bfloat16)

    Zp, S4 = pl.pallas_call(
        _conv_kernel,
        grid=(T, N // NB),
        in_specs=[
            pl.BlockSpec((1, NB, F, A, L), lambda t, nb: (t, nb, 0, 0, 0)),
            pl.BlockSpec((256, 256), lambda t, nb: (0, 0)),
        ],
        out_specs=[
            pl.BlockSpec((1, F, UU, NB, PP), lambda t, nb: (t, 0, 0, nb, 0)),
            pl.BlockSpec((1, 1, NB, UU), lambda t, nb: (t, nb, 0, 0)),
        ],
        out_shape=[
            jax.ShapeDtypeStruct((T, F, UU, N, PP), jnp.float32),
            jax.ShapeDtypeStruct((T, N // NB, NB, UU), jnp.float32),
        ],
        scratch_shapes=[pltpu.VMEM((256, PP + KB), jnp.bfloat16)
                        for _ in range(4)],
        compiler_params=pltpu.CompilerParams(
            dimension_semantics=("parallel", "parallel")),
    )(Xt, RbT)

    S = S4.reshape(T, N, UU)
    Z = jnp.transpose(Zp, (0, 3, 1, 4, 2))
    return (S, R, Z)


# speedup vs baseline: 1.9206x; 1.0791x over previous
import jax
import jax.numpy as jnp
from jax.experimental import pallas as pl
from jax.experimental.pallas import tpu as pltpu

KTAPS = 20
KB = 10
NG = 2
ROWPAD = 24
AA = 21
UU = 100
LL = 334
PP = LL - KTAPS + 1
FF = 6
NB = 8


def _prep_kernel(pl_ref, q_ref, r_ref, rbt_ref):
    pv = pl_ref[...]
    q = q_ref[...]
    meanq = jnp.mean(q)
    eps = jnp.exp(-jnp.log(1.0 / meanq))
    m = jnp.max(pv, axis=1, keepdims=True)
    e = jnp.exp(pv - m)
    p = e / jnp.sum(e, axis=1, keepdims=True)
    ratio = jnp.maximum(p / q, eps)
    rv = jnp.log(ratio)
    r_ref[...] = rv
    rbt_ref[...] = jnp.zeros((256, 256), jnp.bfloat16)
    for k1 in range(NG):
        for k2 in range(KB):
            rbt_ref[k1 * 128:k1 * 128 + UU,
                    k2 * ROWPAD:k2 * ROWPAD + AA] = (
                rv[k1 * KB + k2].astype(jnp.bfloat16).T)


def _conv_kernel(x_ref, rbt_ref, z_ref, s_ref, *cols):
    for c in cols:
        c[...] = jnp.zeros_like(c)
    rbt = rbt_ref[...]

    def build(i):
        j, r = divmod(i, FF)
        xt = x_ref[0, j, r]
        col = cols[i % len(cols)]
        for k2 in range(KB):
            col[k2 * ROWPAD:k2 * ROWPAD + AA, :] = xt[:, k2:k2 + PP + KB]

    build(0)
    build(1)
    smax = None
    for i in range(0, NB * FF, 2):
        j, r = divmod(i, FF)
        yt0 = jnp.dot(rbt, cols[i % len(cols)][...],
                      preferred_element_type=jnp.float32)
        yt1 = jnp.dot(rbt, cols[(i + 1) % len(cols)][...],
                      preferred_element_type=jnp.float32)
        if i + 2 < NB * FF:
            build(i + 2)
        if i + 3 < NB * FF:
            build(i + 3)
        for d, yt in ((0, yt0), (1, yt1)):
            zt = yt[0:UU, 0:PP] + yt[128:128 + UU, KB:KB + PP]
            z_ref[0, r + d, :, j, :] = zt
            m = jnp.max(zt, axis=1)
            smax = m if r + d == 0 else jnp.maximum(smax, m)
        if r + 1 == FF - 1:
            s_ref[0, 0, j, :] = smax


def kernel(X, P_logit, Q):
    T, N, F, L, A = X.shape

    R, RbT = pl.pallas_call(
        _prep_kernel,
        out_shape=[jax.ShapeDtypeStruct((KTAPS, A, UU), jnp.float32),
                   jax.ShapeDtypeStruct((256, 256), jnp.bfloat16)],
    )(P_logit, Q.reshape(1, A, 1))

    Xt = jnp.transpose(X, (0, 1, 2, 4, 3)).astype(jnp.bfloat16)

    Zp, S4 = pl.pallas_call(
        _conv_kernel,
        grid=(T, N // NB),
        in_specs=[
            pl.BlockSpec((1, NB, F, A, L), lambda t, nb: (t, nb, 0, 0, 0)),
            pl.BlockSpec((256, 256), lambda t, nb: (0, 0)),
        ],
        out_specs=[
            pl.BlockSpec((1, F, UU, NB, PP), lambda t, nb: (t, 0, 0, nb, 0)),
            pl.BlockSpec((1, 1, NB, UU), lambda t, nb: (t, nb, 0, 0)),
        ],
        out_shape=[
            jax.ShapeDtypeStruct((T, F, UU, N, PP), jnp.float32),
            jax.ShapeDtypeStruct((T, N // NB, NB, UU), jnp.float32),
        ],
        scratch_shapes=[pltpu.VMEM((256, PP + KB), jnp.bfloat16)
                        for _ in range(4)],
        compiler_params=pltpu.CompilerParams(
            dimension_semantics=("parallel", "parallel")),
    )(Xt, RbT)

    S = S4.reshape(T, N, UU)
    Z = jnp.transpose(Zp, (0, 3, 1, 4, 2))
    return (S, R, Z)
